# baseline (speedup 1.0000x reference)
import functools

import jax
import jax.numpy as jnp
from jax import lax
from jax.experimental import pallas as pl
from jax.experimental.pallas import tpu as pltpu
from jax.experimental.pallas import tpu_sc as plsc

_D = 128
_B = 512
_NW = 32
_BW = _B // _NW


def _sc_partials(class_emb, rel_emb, nf1, nf2, nf3):
    mesh = plsc.VectorSubcoreMesh(core_axis_name="c", subcore_axis_name="s")
    f32 = jnp.float32
    i32 = jnp.int32

    @functools.partial(
        pl.kernel,
        mesh=mesh,
        compiler_params=pltpu.CompilerParams(
            use_tc_tiling_on_sc=False, needs_layout_passes=False),
        out_type=jax.ShapeDtypeStruct((_NW, 4 * _BW), f32),
        scratch_types=(
            pltpu.VMEM((_BW, 2), i32),
            pltpu.VMEM((_BW, 3), i32),
            pltpu.VMEM((_BW, 3), i32),
            pltpu.VMEM((7 * _BW,), i32),
            pltpu.VMEM((7 * _BW, 2 * _D), f32),
            pltpu.VMEM((_BW, _D), f32),
            pltpu.VMEM((4 * _BW,), f32),
            pltpu.SemaphoreType.DMA,
            pltpu.SemaphoreType.DMA,
            pltpu.SemaphoreType.DMA,
            pltpu.SemaphoreType.DMA,
            pltpu.SemaphoreType.DMA,
        ),
    )
    def k(class_hbm, rel_hbm, nf1_hbm, nf2_hbm, nf3_hbm, out_hbm,
          xi1, xi2, xi3, xcomb, vcls, vrel, sall,
          s1m, s2m, s3m, scm, srm):
        wid = lax.axis_index("s") * 2 + lax.axis_index("c")
        base = wid * _BW
        sl = pl.ds(base, _BW)

        cpi1 = pltpu.async_copy(nf1_hbm.at[sl], xi1, s1m)
        cpi2 = pltpu.async_copy(nf2_hbm.at[sl], xi2, s2m)
        cpi3 = pltpu.async_copy(nf3_hbm.at[sl], xi3, s3m)
        cpi1.wait()
        cpi2.wait()
        cpi3.wait()

        lanes = lax.broadcasted_iota(i32, (_BW,), 0)

        def getcol(ref, c):
            return plsc.load_gather(ref, [lanes, jnp.full((_BW,), c, i32)])

        xcomb[pl.ds(0 * _BW, _BW)] = getcol(xi1, 0)
        xcomb[pl.ds(1 * _BW, _BW)] = getcol(xi1, 1)
        xcomb[pl.ds(2 * _BW, _BW)] = getcol(xi2, 0)
        xcomb[pl.ds(3 * _BW, _BW)] = getcol(xi2, 1)
        xcomb[pl.ds(4 * _BW, _BW)] = getcol(xi2, 2)
        xcomb[pl.ds(5 * _BW, _BW)] = getcol(xi3, 0)
        xcomb[pl.ds(6 * _BW, _BW)] = getcol(xi3, 2)
        i3r = getcol(xi3, 1)

        cpc = pltpu.async_copy(class_hbm.at[xcomb], vcls, scm)
        cpr = pltpu.async_copy(rel_hbm.at[i3r], vrel, srm)
        cpc.wait()
        cpr.wait()

        def col(ref, lvec, dvec):
            return plsc.load_gather(ref, [lvec, dvec])

        l1c = lanes
        l1d = lanes + _BW
        l2c = lanes + 2 * _BW
        l2d = lanes + 3 * _BW
        l2e = lanes + 4 * _BW
        l3c = lanes + 5 * _BW
        l3d = lanes + 6 * _BW

        def body1(d, acc):
            dc = jnp.full((_BW,), d, i32)
            dc2 = dc + _D
            c1 = col(vcls, l1c, dc)
            cr = col(vcls, l1c, dc2)
            d1 = col(vcls, l1d, dc)
            dr = col(vcls, l1d, dc2)
            t = jnp.maximum(jnp.abs(c1 - d1) + jnp.abs(cr) - jnp.abs(dr), 0.0)
            return acc + t * t

        sall[pl.ds(0, _BW)] = plsc.parallel_loop(
            0, _D, unroll=8, carry=jnp.zeros((_BW,), f32))(body1)

        def body2(d, accs):
            aa, ab = accs
            dc = jnp.full((_BW,), d, i32)
            dc2 = dc + _D
            c1 = col(vcls, l2c, dc)
            c2 = jnp.abs(col(vcls, l2c, dc2))
            d1 = col(vcls, l2d, dc)
            d2 = jnp.abs(col(vcls, l2d, dc2))
            e1 = col(vcls, l2e, dc)
            e2 = jnp.abs(col(vcls, l2e, dc2))
            start = jnp.maximum(c1 - c2, d1 - d2)
            end = jnp.minimum(c1 + c2, d1 + d2)
            diff = start - end
            cen = (start + end) * 0.5
            t1 = jnp.maximum(jnp.abs(cen - e1) + jnp.abs(diff) * 0.5 - e2, 0.0)
            t2 = jnp.maximum(diff, 0.0)
            return (aa + t1 * t1, ab + t2 * t2)

        ra, rb = plsc.parallel_loop(
            0, _D, unroll=8,
            carry=(jnp.zeros((_BW,), f32), jnp.zeros((_BW,), f32)))(body2)
        sall[pl.ds(_BW, _BW)] = ra
        sall[pl.ds(2 * _BW, _BW)] = rb

        def body3(d, acc):
            dc = jnp.full((_BW,), d, i32)
            dc2 = dc + _D
            c1 = col(vcls, l3c, dc)
            cr = col(vcls, l3c, dc2)
            d1 = col(vcls, l3d, dc)
            dr = col(vcls, l3d, dc2)
            r = col(vrel, lanes, dc)
            t = jnp.maximum(jnp.abs(c1 + r - d1) + jnp.abs(cr) - jnp.abs(dr), 0.0)
            return acc + t * t

        sall[pl.ds(3 * _BW, _BW)] = plsc.parallel_loop(
            0, _D, unroll=8, carry=jnp.zeros((_BW,), f32))(body3)

        pltpu.sync_copy(sall, out_hbm.at[wid])

    return k(class_emb, rel_emb, nf1, nf2, nf3)


def _reduce_body(p_ref, o_ref):
    p = p_ref[...]
    s1 = p[:, 0 * _BW:1 * _BW]
    sa = p[:, 1 * _BW:2 * _BW]
    sb = p[:, 2 * _BW:3 * _BW]
    s3 = p[:, 3 * _BW:4 * _BW]
    inv = 1.0 / _B
    loss = (jnp.sum(s1) + jnp.sum(sa) + jnp.sum(sb) + jnp.sum(s3)) * inv \
        + 2.0 * (jnp.sum(jnp.sqrt(sa)) * inv) * (jnp.sum(jnp.sqrt(sb)) * inv)
    o_ref[...] = jnp.full((1, 1), loss, jnp.float32)


def kernel(class_emb, rel_emb, nf1, nf2, nf3):
    i32 = jnp.int32
    partials = _sc_partials(
        class_emb, rel_emb,
        nf1.astype(i32), nf2.astype(i32), nf3.astype(i32))
    out = pl.pallas_call(
        _reduce_body,
        out_shape=jax.ShapeDtypeStruct((1, 1), jnp.float32),
    )(partials)
    return out[0, 0]


# speedup vs baseline: 1.4410x; 1.4410x over previous
import functools

import jax
import jax.numpy as jnp
from jax import lax
from jax.experimental import pallas as pl
from jax.experimental.pallas import tpu as pltpu
from jax.experimental.pallas import tpu_sc as plsc

_D = 128
_B = 512
_NW = 32
_BW = _B // _NW
_L = 16


def _sc_partials(class_emb, rel_emb, nf1, nf2, nf3):
    mesh = plsc.VectorSubcoreMesh(core_axis_name="c", subcore_axis_name="s")
    f32 = jnp.float32
    i32 = jnp.int32

    @functools.partial(
        pl.kernel,
        mesh=mesh,
        compiler_params=pltpu.CompilerParams(
            use_tc_tiling_on_sc=False, needs_layout_passes=False),
        out_type=jax.ShapeDtypeStruct((_NW, 4 * _BW), f32),
        scratch_types=(
            pltpu.VMEM((_BW, 2), i32),
            pltpu.VMEM((_BW, 3), i32),
            pltpu.VMEM((_BW, 3), i32),
            pltpu.VMEM((7 * _BW,), i32),
            pltpu.VMEM((7 * _BW, 2 * _D), f32),
            pltpu.VMEM((_BW, _D), f32),
            pltpu.VMEM((_BW, _L + 1), f32),
            pltpu.VMEM((_BW, _L + 1), f32),
            pltpu.VMEM((4 * _BW,), f32),
            pltpu.SemaphoreType.DMA,
            pltpu.SemaphoreType.DMA,
            pltpu.SemaphoreType.DMA,
            pltpu.SemaphoreType.DMA,
            pltpu.SemaphoreType.DMA,
        ),
    )
    def k(class_hbm, rel_hbm, nf1_hbm, nf2_hbm, nf3_hbm, out_hbm,
          xi1, xi2, xi3, xcomb, vcls, vrel, accma, accmb, sall,
          s1m, s2m, s3m, scm, srm):
        wid = lax.axis_index("s") * 2 + lax.axis_index("c")
        base = wid * _BW
        sl = pl.ds(base, _BW)

        cpi1 = pltpu.async_copy(nf1_hbm.at[sl], xi1, s1m)
        cpi2 = pltpu.async_copy(nf2_hbm.at[sl], xi2, s2m)
        cpi3 = pltpu.async_copy(nf3_hbm.at[sl], xi3, s3m)
        cpi1.wait()
        cpi2.wait()
        cpi3.wait()

        lanes = lax.broadcasted_iota(i32, (_BW,), 0)

        def getcol(ref, c):
            return plsc.load_gather(ref, [lanes, jnp.full((_BW,), c, i32)])

        xcomb[pl.ds(0 * _BW, _BW)] = getcol(xi1, 0)
        xcomb[pl.ds(1 * _BW, _BW)] = getcol(xi1, 1)
        xcomb[pl.ds(2 * _BW, _BW)] = getcol(xi2, 0)
        xcomb[pl.ds(3 * _BW, _BW)] = getcol(xi2, 1)
        xcomb[pl.ds(4 * _BW, _BW)] = getcol(xi2, 2)
        xcomb[pl.ds(5 * _BW, _BW)] = getcol(xi3, 0)
        xcomb[pl.ds(6 * _BW, _BW)] = getcol(xi3, 2)
        i3r = getcol(xi3, 1)

        cpc = pltpu.async_copy(class_hbm.at[xcomb], vcls, scm)
        cpr = pltpu.async_copy(rel_hbm.at[i3r], vrel, srm)
        cpc.wait()
        cpr.wait()

        def body1(r, acc):
            rd = r + _BW
            for j in range(_D // _L):
                lo = pl.ds(_L * j, _L)
                hi = pl.ds(_D + _L * j, _L)
                c1 = vcls[r, lo]
                cr = vcls[r, hi]
                d1 = vcls[rd, lo]
                dr = vcls[rd, hi]
                t = jnp.maximum(
                    jnp.abs(c1 - d1) + jnp.abs(cr) - jnp.abs(dr), 0.0)
                acc = acc + t * t
            return acc

        acc1 = plsc.parallel_loop(
            0, _BW, unroll=2, carry=jnp.zeros((_L,), f32))(body1)
        sall[pl.ds(0, _L)] = acc1

        @plsc.parallel_loop(0, _BW, unroll=2)
        def body2(r):
            rc = r + 2 * _BW
            rd = r + 3 * _BW
            re = r + 4 * _BW
            aa = jnp.zeros((_L,), f32)
            ab = jnp.zeros((_L,), f32)
            for j in range(_D // _L):
                lo = pl.ds(_L * j, _L)
                hi = pl.ds(_D + _L * j, _L)
                c1 = vcls[rc, lo]
                c2 = jnp.abs(vcls[rc, hi])
                d1 = vcls[rd, lo]
                d2 = jnp.abs(vcls[rd, hi])
                e1 = vcls[re, lo]
                e2 = jnp.abs(vcls[re, hi])
                start = jnp.maximum(c1 - c2, d1 - d2)
                end = jnp.minimum(c1 + c2, d1 + d2)
                diff = start - end
                cen = (start + end) * 0.5
                t1 = jnp.maximum(
                    jnp.abs(cen - e1) + jnp.abs(diff) * 0.5 - e2, 0.0)
                t2 = jnp.maximum(diff, 0.0)
                aa = aa + t1 * t1
                ab = ab + t2 * t2
            accma[r, pl.ds(0, _L)] = aa
            accmb[r, pl.ds(0, _L)] = ab

        def body3(r, acc):
            rc = r + 5 * _BW
            rd = r + 6 * _BW
            for j in range(_D // _L):
                lo = pl.ds(_L * j, _L)
                hi = pl.ds(_D + _L * j, _L)
                c1 = vcls[rc, lo]
                cr = vcls[rc, hi]
                d1 = vcls[rd, lo]
                dr = vcls[rd, hi]
                rr = vrel[r, lo]
                t = jnp.maximum(
                    jnp.abs(c1 + rr - d1) + jnp.abs(cr) - jnp.abs(dr), 0.0)
                acc = acc + t * t
            return acc

        acc3 = plsc.parallel_loop(
            0, _BW, unroll=2, carry=jnp.zeros((_L,), f32))(body3)
        sall[pl.ds(3 * _L, _L)] = acc3

        sa = jnp.zeros((_L,), f32)
        sb = jnp.zeros((_L,), f32)
        for c in range(_L):
            cc = jnp.full((_BW,), c, i32)
            sa = sa + plsc.load_gather(accma, [lanes, cc])
            sb = sb + plsc.load_gather(accmb, [lanes, cc])
        sall[pl.ds(_L, _L)] = sa
        sall[pl.ds(2 * _L, _L)] = sb

        pltpu.sync_copy(sall, out_hbm.at[wid])

    return k(class_emb, rel_emb, nf1, nf2, nf3)


def _reduce_body(p_ref, o_ref):
    p = p_ref[...]
    s1 = p[:, 0 * _BW:1 * _BW]
    sa = p[:, 1 * _BW:2 * _BW]
    sb = p[:, 2 * _BW:3 * _BW]
    s3 = p[:, 3 * _BW:4 * _BW]
    inv = 1.0 / _B
    loss = (jnp.sum(s1) + jnp.sum(sa) + jnp.sum(sb) + jnp.sum(s3)) * inv \
        + 2.0 * (jnp.sum(jnp.sqrt(sa)) * inv) * (jnp.sum(jnp.sqrt(sb)) * inv)
    o_ref[...] = jnp.full((1, 1), loss, jnp.float32)


def kernel(class_emb, rel_emb, nf1, nf2, nf3):
    i32 = jnp.int32
    partials = _sc_partials(
        class_emb, rel_emb,
        nf1.astype(i32), nf2.astype(i32), nf3.astype(i32))
    out = pl.pallas_call(
        _reduce_body,
        out_shape=jax.ShapeDtypeStruct((1, 1), jnp.float32),
    )(partials)
    return out[0, 0]


# speedup vs baseline: 1.4505x; 1.0066x over previous
import functools

import jax
import jax.numpy as jnp
from jax import lax
from jax.experimental import pallas as pl
from jax.experimental.pallas import tpu as pltpu
from jax.experimental.pallas import tpu_sc as plsc

_D = 128
_B = 512
_NW = 32
_BW = _B // _NW
_L = 16


def _sc_partials(class_emb, rel_emb, nf1, nf2, nf3):
    mesh = plsc.VectorSubcoreMesh(core_axis_name="c", subcore_axis_name="s")
    f32 = jnp.float32
    i32 = jnp.int32

    @functools.partial(
        pl.kernel,
        mesh=mesh,
        compiler_params=pltpu.CompilerParams(
            use_tc_tiling_on_sc=False, needs_layout_passes=False),
        out_type=jax.ShapeDtypeStruct((_NW, 4 * _BW), f32),
        scratch_types=(
            pltpu.VMEM((2 * _BW,), i32),
            pltpu.VMEM((3 * _BW,), i32),
            pltpu.VMEM((3 * _BW,), i32),
            pltpu.VMEM((7 * _BW,), i32),
            pltpu.VMEM((7 * _BW, 2 * _D), f32),
            pltpu.VMEM((_BW, _D), f32),
            pltpu.VMEM((_BW, _L + 1), f32),
            pltpu.VMEM((_BW, _L + 1), f32),
            pltpu.VMEM((4 * _BW,), f32),
            pltpu.SemaphoreType.DMA,
            pltpu.SemaphoreType.DMA,
            pltpu.SemaphoreType.DMA,
            pltpu.SemaphoreType.DMA,
            pltpu.SemaphoreType.DMA,
        ),
    )
    def k(class_hbm, rel_hbm, nf1_hbm, nf2_hbm, nf3_hbm, out_hbm,
          xi1, xi2, xi3, xcomb, vcls, vrel, accma, accmb, sall,
          s1m, s2m, s3m, scm, srm):
        wid = lax.axis_index("s") * 2 + lax.axis_index("c")
        base = wid * _BW

        cpi1 = pltpu.async_copy(nf1_hbm.at[pl.ds(2 * base, 2 * _BW)], xi1, s1m)
        cpi2 = pltpu.async_copy(nf2_hbm.at[pl.ds(3 * base, 3 * _BW)], xi2, s2m)
        cpi3 = pltpu.async_copy(nf3_hbm.at[pl.ds(3 * base, 3 * _BW)], xi3, s3m)
        cpi1.wait()
        cpi2.wait()
        cpi3.wait()

        lanes = lax.broadcasted_iota(i32, (_BW,), 0)

        def getcol(ref, stride, c):
            return plsc.load_gather(ref, [lanes * stride + c])

        xcomb[pl.ds(0 * _BW, _BW)] = getcol(xi1, 2, 0)
        xcomb[pl.ds(1 * _BW, _BW)] = getcol(xi1, 2, 1)
        xcomb[pl.ds(2 * _BW, _BW)] = getcol(xi2, 3, 0)
        xcomb[pl.ds(3 * _BW, _BW)] = getcol(xi2, 3, 1)
        xcomb[pl.ds(4 * _BW, _BW)] = getcol(xi2, 3, 2)
        xcomb[pl.ds(5 * _BW, _BW)] = getcol(xi3, 3, 0)
        xcomb[pl.ds(6 * _BW, _BW)] = getcol(xi3, 3, 2)
        i3r = getcol(xi3, 3, 1)

        cpc = pltpu.async_copy(class_hbm.at[xcomb], vcls, scm)
        cpr = pltpu.async_copy(rel_hbm.at[i3r], vrel, srm)
        cpc.wait()
        cpr.wait()

        def body1(r, acc):
            rd = r + _BW
            for j in range(_D // _L):
                lo = pl.ds(_L * j, _L)
                hi = pl.ds(_D + _L * j, _L)
                c1 = vcls[r, lo]
                cr = vcls[r, hi]
                d1 = vcls[rd, lo]
                dr = vcls[rd, hi]
                t = jnp.maximum(
                    jnp.abs(c1 - d1) + jnp.abs(cr) - jnp.abs(dr), 0.0)
                acc = acc + t * t
            return acc

        acc1 = plsc.parallel_loop(
            0, _BW, unroll=1, carry=jnp.zeros((_L,), f32))(body1)
        sall[pl.ds(0, _L)] = acc1

        @plsc.parallel_loop(0, _BW, unroll=1)
        def body2(r):
            rc = r + 2 * _BW
            rd = r + 3 * _BW
            re = r + 4 * _BW
            aa = jnp.zeros((_L,), f32)
            ab = jnp.zeros((_L,), f32)
            for j in range(_D // _L):
                lo = pl.ds(_L * j, _L)
                hi = pl.ds(_D + _L * j, _L)
                c1 = vcls[rc, lo]
                c2 = jnp.abs(vcls[rc, hi])
                d1 = vcls[rd, lo]
                d2 = jnp.abs(vcls[rd, hi])
                e1 = vcls[re, lo]
                e2 = jnp.abs(vcls[re, hi])
                start = jnp.maximum(c1 - c2, d1 - d2)
                end = jnp.minimum(c1 + c2, d1 + d2)
                diff = start - end
                cen = (start + end) * 0.5
                t1 = jnp.maximum(
                    jnp.abs(cen - e1) + jnp.abs(diff) * 0.5 - e2, 0.0)
                t2 = jnp.maximum(diff, 0.0)
                aa = aa + t1 * t1
                ab = ab + t2 * t2
            accma[r, pl.ds(0, _L)] = aa
            accmb[r, pl.ds(0, _L)] = ab

        def body3(r, acc):
            rc = r + 5 * _BW
            rd = r + 6 * _BW
            for j in range(_D // _L):
                lo = pl.ds(_L * j, _L)
                hi = pl.ds(_D + _L * j, _L)
                c1 = vcls[rc, lo]
                cr = vcls[rc, hi]
                d1 = vcls[rd, lo]
                dr = vcls[rd, hi]
                rr = vrel[r, lo]
                t = jnp.maximum(
                    jnp.abs(c1 + rr - d1) + jnp.abs(cr) - jnp.abs(dr), 0.0)
                acc = acc + t * t
            return acc

        acc3 = plsc.parallel_loop(
            0, _BW, unroll=1, carry=jnp.zeros((_L,), f32))(body3)
        sall[pl.ds(3 * _L, _L)] = acc3

        sa = jnp.zeros((_L,), f32)
        sb = jnp.zeros((_L,), f32)
        for c in range(_L):
            cc = jnp.full((_BW,), c, i32)
            sa = sa + plsc.load_gather(accma, [lanes, cc])
            sb = sb + plsc.load_gather(accmb, [lanes, cc])
        sall[pl.ds(_L, _L)] = sa
        sall[pl.ds(2 * _L, _L)] = sb

        pltpu.sync_copy(sall, out_hbm.at[wid])

    return k(class_emb, rel_emb, nf1, nf2, nf3)


def _reduce_body(p_ref, o_ref):
    p = p_ref[...]
    s1 = p[:, 0 * _BW:1 * _BW]
    sa = p[:, 1 * _BW:2 * _BW]
    sb = p[:, 2 * _BW:3 * _BW]
    s3 = p[:, 3 * _BW:4 * _BW]
    inv = 1.0 / _B
    loss = (jnp.sum(s1) + jnp.sum(sa) + jnp.sum(sb) + jnp.sum(s3)) * inv \
        + 2.0 * (jnp.sum(jnp.sqrt(sa)) * inv) * (jnp.sum(jnp.sqrt(sb)) * inv)
    o_ref[...] = jnp.full((1, 1), loss, jnp.float32)


def kernel(class_emb, rel_emb, nf1, nf2, nf3):
    i32 = jnp.int32
    partials = _sc_partials(
        class_emb, rel_emb,
        nf1.astype(i32).reshape(-1), nf2.astype(i32).reshape(-1),
        nf3.astype(i32).reshape(-1))
    out = pl.pallas_call(
        _reduce_body,
        out_shape=jax.ShapeDtypeStruct((1, 1), jnp.float32),
    )(partials)
    return out[0, 0]


# speedup vs baseline: 1.6267x; 1.1214x over previous
import functools

import jax
import jax.numpy as jnp
from jax import lax
from jax.experimental import pallas as pl
from jax.experimental.pallas import tpu as pltpu
from jax.experimental.pallas import tpu_sc as plsc

_D = 128
_B = 512
_NW = 32
_BW = _B // _NW
_L = 16


def _sc_partials(table, nfall):
    mesh = plsc.VectorSubcoreMesh(core_axis_name="c", subcore_axis_name="s")
    f32 = jnp.float32
    i32 = jnp.int32

    @functools.partial(
        pl.kernel,
        mesh=mesh,
        compiler_params=pltpu.CompilerParams(
            use_tc_tiling_on_sc=False, needs_layout_passes=False),
        out_type=jax.ShapeDtypeStruct((4 * _L * _NW,), f32),
        scratch_types=(
            pltpu.VMEM((8 * _BW,), i32),
            pltpu.VMEM((8 * _BW,), i32),
            pltpu.VMEM((8 * _BW, 2 * _D), f32),
            pltpu.VMEM((_BW * (_L + 1),), f32),
            pltpu.VMEM((_BW * (_L + 1),), f32),
            pltpu.VMEM((4 * _L,), f32),
            pltpu.SemaphoreType.DMA,
            pltpu.SemaphoreType.DMA,
        ),
    )
    def k(table_hbm, nf_hbm, out_hbm,
          xi, xcomb, vcls, accma, accmb, sall,
          sim, scm):
        wid = lax.axis_index("s") * 2 + lax.axis_index("c")
        base = wid * _BW

        pltpu.async_copy(nf_hbm.at[pl.ds(8 * base, 8 * _BW)], xi, sim).wait()

        lanes = lax.broadcasted_iota(i32, (_BW,), 0)
        lanes8 = lanes * 8

        def getcol(c):
            return plsc.load_gather(xi, [lanes8 + c])

        for c in range(8):
            xcomb[pl.ds(c * _BW, _BW)] = getcol(c)

        pltpu.async_copy(table_hbm.at[xcomb], vcls, scm).wait()

        def body1(r, acc):
            rd = r + _BW
            for j in range(_D // _L):
                lo = pl.ds(_L * j, _L)
                hi = pl.ds(_D + _L * j, _L)
                c1 = vcls[r, lo]
                cr = vcls[r, hi]
                d1 = vcls[rd, lo]
                dr = vcls[rd, hi]
                t = jnp.maximum(
                    jnp.abs(c1 - d1) + jnp.abs(cr) - jnp.abs(dr), 0.0)
                acc = acc + t * t
            return acc

        acc1 = plsc.parallel_loop(
            0, _BW, unroll=1, carry=jnp.zeros((_L,), f32))(body1)
        sall[pl.ds(0, _L)] = acc1

        @plsc.parallel_loop(0, _BW, unroll=1)
        def body2(r):
            rc = r + 2 * _BW
            rd = r + 3 * _BW
            re = r + 4 * _BW
            aa = jnp.zeros((_L,), f32)
            ab = jnp.zeros((_L,), f32)
            for j in range(_D // _L):
                lo = pl.ds(_L * j, _L)
                hi = pl.ds(_D + _L * j, _L)
                c1 = vcls[rc, lo]
                c2 = jnp.abs(vcls[rc, hi])
                d1 = vcls[rd, lo]
                d2 = jnp.abs(vcls[rd, hi])
                e1 = vcls[re, lo]
                e2 = jnp.abs(vcls[re, hi])
                start = jnp.maximum(c1 - c2, d1 - d2)
                end = jnp.minimum(c1 + c2, d1 + d2)
                diff = start - end
                cen = (start + end) * 0.5
                t1 = jnp.maximum(
                    jnp.abs(cen - e1) + jnp.abs(diff) * 0.5 - e2, 0.0)
                t2 = jnp.maximum(diff, 0.0)
                aa = aa + t1 * t1
                ab = ab + t2 * t2
            accma[pl.ds(r * (_L + 1), _L)] = aa
            accmb[pl.ds(r * (_L + 1), _L)] = ab

        def body3(r, acc):
            rc = r + 5 * _BW
            rd = r + 6 * _BW
            rr_ = r + 7 * _BW
            for j in range(_D // _L):
                lo = pl.ds(_L * j, _L)
                hi = pl.ds(_D + _L * j, _L)
                c1 = vcls[rc, lo]
                cr = vcls[rc, hi]
                d1 = vcls[rd, lo]
                dr = vcls[rd, hi]
                rr = vcls[rr_, lo]
                t = jnp.maximum(
                    jnp.abs(c1 + rr - d1) + jnp.abs(cr) - jnp.abs(dr), 0.0)
                acc = acc + t * t
            return acc

        acc3 = plsc.parallel_loop(
            0, _BW, unroll=1, carry=jnp.zeros((_L,), f32))(body3)
        sall[pl.ds(3 * _L, _L)] = acc3

        sa = jnp.zeros((_L,), f32)
        sb = jnp.zeros((_L,), f32)
        lanes17 = lanes * (_L + 1)
        for c in range(_L):
            sa = sa + plsc.load_gather(accma, [lanes17 + c])
            sb = sb + plsc.load_gather(accmb, [lanes17 + c])
        sall[pl.ds(_L, _L)] = sa
        sall[pl.ds(2 * _L, _L)] = sb

        pltpu.sync_copy(sall, out_hbm.at[pl.ds(wid * 4 * _L, 4 * _L)])

    return k(table, nfall)


def _reduce_body(p_ref, o_ref):
    p = p_ref[...]

    def blk(k):
        return jnp.sum(p[:, k * _L:(k + 1) * _L]) \
            + jnp.sum(p[:, 64 + k * _L:64 + (k + 1) * _L])

    def blk_sqrt(k):
        return jnp.sum(jnp.sqrt(p[:, k * _L:(k + 1) * _L])) \
            + jnp.sum(jnp.sqrt(p[:, 64 + k * _L:64 + (k + 1) * _L]))

    inv = 1.0 / _B
    loss = (blk(0) + blk(1) + blk(2) + blk(3)) * inv \
        + 2.0 * (blk_sqrt(1) * inv) * (blk_sqrt(2) * inv)
    o_ref[...] = jnp.full((1, 1), loss, jnp.float32)


def kernel(class_emb, rel_emb, nf1, nf2, nf3):
    i32 = jnp.int32
    table = jnp.concatenate(
        [class_emb, jnp.pad(rel_emb, ((0, 0), (0, _D)))], axis=0)
    nfall = jnp.stack(
        [nf1[:, 0].astype(i32), nf1[:, 1].astype(i32),
         nf2[:, 0].astype(i32), nf2[:, 1].astype(i32), nf2[:, 2].astype(i32),
         nf3[:, 0].astype(i32), nf3[:, 2].astype(i32),
         nf3[:, 1].astype(i32) + 1000],
        axis=1).reshape(-1)
    partials = _sc_partials(table, nfall)
    out = pl.pallas_call(
        _reduce_body,
        out_shape=jax.ShapeDtypeStruct((1, 1), jnp.float32),
    )(partials.reshape(_L, 2 * 64))
    return out[0, 0]


# speedup vs baseline: 1.6273x; 1.0004x over previous
import functools

import jax
import jax.numpy as jnp
from jax import lax
from jax.experimental import pallas as pl
from jax.experimental.pallas import tpu as pltpu
from jax.experimental.pallas import tpu_sc as plsc

_D = 128
_B = 512
_NW = 32
_BW = _B // _NW
_L = 16


def _sc_partials(table, nfall):
    mesh = plsc.VectorSubcoreMesh(core_axis_name="c", subcore_axis_name="s")
    f32 = jnp.float32
    i32 = jnp.int32

    @functools.partial(
        pl.kernel,
        mesh=mesh,
        compiler_params=pltpu.CompilerParams(
            use_tc_tiling_on_sc=False, needs_layout_passes=False),
        out_type=jax.ShapeDtypeStruct((4 * _L * _NW,), f32),
        scratch_types=(
            pltpu.VMEM((8 * _BW,), i32),
            pltpu.VMEM((8 * _BW,), i32),
            pltpu.VMEM((8 * _BW, 2 * _D), f32),
            pltpu.VMEM((_BW * (_L + 1),), f32),
            pltpu.VMEM((_BW * (_L + 1),), f32),
            pltpu.VMEM((4 * _L,), f32),
            pltpu.SemaphoreType.DMA,
            pltpu.SemaphoreType.DMA,
        ),
    )
    def k(table_hbm, nf_hbm, out_hbm,
          xi, xcomb, vcls, accma, accmb, sall,
          sim, scm):
        wid = lax.axis_index("s") * 2 + lax.axis_index("c")
        base = wid * _BW

        pltpu.async_copy(nf_hbm.at[pl.ds(8 * base, 8 * _BW)], xi, sim).wait()

        lanes = lax.broadcasted_iota(i32, (_BW,), 0)
        lanes8 = lanes * 8

        def getcol(c):
            return plsc.load_gather(xi, [lanes8 + c])

        for c in range(8):
            xcomb[pl.ds(c * _BW, _BW)] = getcol(c)

        pltpu.async_copy(table_hbm.at[xcomb], vcls, scm).wait()

        def body1(r, acc):
            rd = r + _BW

            def chunk1(j, a):
                lo = pl.ds(_L * j, _L)
                hi = pl.ds(_D + _L * j, _L)
                c1 = vcls[r, lo]
                cr = vcls[r, hi]
                d1 = vcls[rd, lo]
                dr = vcls[rd, hi]
                t = jnp.maximum(
                    jnp.abs(c1 - d1) + jnp.abs(cr) - jnp.abs(dr), 0.0)
                return a + t * t

            return lax.fori_loop(0, _D // _L, chunk1, acc)

        acc1 = plsc.parallel_loop(
            0, _BW, unroll=1, carry=jnp.zeros((_L,), f32))(body1)
        sall[pl.ds(0, _L)] = acc1

        @plsc.parallel_loop(0, _BW, unroll=1)
        def body2(r):
            rc = r + 2 * _BW
            rd = r + 3 * _BW
            re = r + 4 * _BW
            def chunk2(j, accs):
                aa, ab = accs
                lo = pl.ds(_L * j, _L)
                hi = pl.ds(_D + _L * j, _L)
                c1 = vcls[rc, lo]
                c2 = jnp.abs(vcls[rc, hi])
                d1 = vcls[rd, lo]
                d2 = jnp.abs(vcls[rd, hi])
                e1 = vcls[re, lo]
                e2 = jnp.abs(vcls[re, hi])
                start = jnp.maximum(c1 - c2, d1 - d2)
                end = jnp.minimum(c1 + c2, d1 + d2)
                diff = start - end
                cen = (start + end) * 0.5
                t1 = jnp.maximum(
                    jnp.abs(cen - e1) + jnp.abs(diff) * 0.5 - e2, 0.0)
                t2 = jnp.maximum(diff, 0.0)
                return (aa + t1 * t1, ab + t2 * t2)

            aa, ab = lax.fori_loop(
                0, _D // _L, chunk2,
                (jnp.zeros((_L,), f32), jnp.zeros((_L,), f32)))
            accma[pl.ds(r * (_L + 1), _L)] = aa
            accmb[pl.ds(r * (_L + 1), _L)] = ab

        def body3(r, acc):
            rc = r + 5 * _BW
            rd = r + 6 * _BW
            rr_ = r + 7 * _BW

            def chunk3(j, a):
                lo = pl.ds(_L * j, _L)
                hi = pl.ds(_D + _L * j, _L)
                c1 = vcls[rc, lo]
                cr = vcls[rc, hi]
                d1 = vcls[rd, lo]
                dr = vcls[rd, hi]
                rr = vcls[rr_, lo]
                t = jnp.maximum(
                    jnp.abs(c1 + rr - d1) + jnp.abs(cr) - jnp.abs(dr), 0.0)
                return a + t * t

            return lax.fori_loop(0, _D // _L, chunk3, acc)

        acc3 = plsc.parallel_loop(
            0, _BW, unroll=1, carry=jnp.zeros((_L,), f32))(body3)
        sall[pl.ds(3 * _L, _L)] = acc3

        sa = jnp.zeros((_L,), f32)
        sb = jnp.zeros((_L,), f32)
        lanes17 = lanes * (_L + 1)
        for c in range(_L):
            sa = sa + plsc.load_gather(accma, [lanes17 + c])
            sb = sb + plsc.load_gather(accmb, [lanes17 + c])
        sall[pl.ds(_L, _L)] = sa
        sall[pl.ds(2 * _L, _L)] = sb

        pltpu.sync_copy(sall, out_hbm.at[pl.ds(wid * 4 * _L, 4 * _L)])

    return k(table, nfall)


def _reduce_body(p_ref, o_ref):
    p = p_ref[...]

    def blk(k):
        return jnp.sum(p[:, k * _L:(k + 1) * _L]) \
            + jnp.sum(p[:, 64 + k * _L:64 + (k + 1) * _L])

    def blk_sqrt(k):
        return jnp.sum(jnp.sqrt(p[:, k * _L:(k + 1) * _L])) \
            + jnp.sum(jnp.sqrt(p[:, 64 + k * _L:64 + (k + 1) * _L]))

    inv = 1.0 / _B
    loss = (blk(0) + blk(1) + blk(2) + blk(3)) * inv \
        + 2.0 * (blk_sqrt(1) * inv) * (blk_sqrt(2) * inv)
    o_ref[...] = jnp.full((1, 1), loss, jnp.float32)


def kernel(class_emb, rel_emb, nf1, nf2, nf3):
    i32 = jnp.int32
    table = jnp.concatenate(
        [class_emb, jnp.pad(rel_emb, ((0, 0), (0, _D)))], axis=0)
    nfall = jnp.stack(
        [nf1[:, 0].astype(i32), nf1[:, 1].astype(i32),
         nf2[:, 0].astype(i32), nf2[:, 1].astype(i32), nf2[:, 2].astype(i32),
         nf3[:, 0].astype(i32), nf3[:, 2].astype(i32),
         nf3[:, 1].astype(i32) + 1000],
        axis=1).reshape(-1)
    partials = _sc_partials(table, nfall)
    out = pl.pallas_call(
        _reduce_body,
        out_shape=jax.ShapeDtypeStruct((1, 1), jnp.float32),
    )(partials.reshape(_L, 2 * 64))
    return out[0, 0]
